# baseline (speedup 1.0000x reference)
import functools

import jax
import jax.numpy as jnp
from jax import lax
from jax.experimental import pallas as pl
from jax.experimental.pallas import tpu as pltpu
from jax.experimental.pallas import tpu_sc as plsc

VOCAB = 1000000
HIDDEN = 32
BATCH = 4096
SEQ = 200


BN = 131072
NBLK = (VOCAB + BN - 1) // BN


def _matvec_body(xa_ref, xb_ref, wa_ref, wb_ref, b_ref, o_ref):
    o_ref[...] = (jnp.sum(xa_ref[...] * wa_ref[...], axis=0)
                  + jnp.sum(xb_ref[...] * wb_ref[...], axis=0)
                  + b_ref[0, 0])


def _compute_t(table_t, ws, b2):
    half = HIDDEN // 2
    return pl.pallas_call(
        _matvec_body,
        grid=(NBLK,),
        in_specs=[
            pl.BlockSpec((half, BN), lambda i: (0, i)),
            pl.BlockSpec((half, BN), lambda i: (1, i)),
            pl.BlockSpec((half, 1), lambda i: (0, 0)),
            pl.BlockSpec((half, 1), lambda i: (1, 0)),
            pl.BlockSpec((1, 1), lambda i: (0, 0)),
        ],
        out_specs=pl.BlockSpec((BN,), lambda i: (i,)),
        out_shape=jax.ShapeDtypeStruct((VOCAB,), jnp.float32),
    )(table_t, table_t, ws, ws, b2)


NUM_WORKERS = 32
SAMP_PER_W = BATCH // NUM_WORKERS
GB = 25
NBATCH = SEQ // GB
NSEM = 3
NGRP = SAMP_PER_W // 16


def _pool_body(ids_hbm, t_hbm, out_hbm, idx_v, vals_v, out_v,
               sem0, sem1, sem2):
    wid = lax.axis_index("s") * 2 + lax.axis_index("c")
    s0 = wid * SAMP_PER_W
    sems = (sem0, sem1, sem2)

    SPLIT = 80
    pltpu.sync_copy(ids_hbm.at[pl.ds(0, SPLIT), pl.ds(s0, SAMP_PER_W)],
                    idx_v.at[pl.ds(0, SPLIT)])

    def fire_batch(k):
        for r in range(k * GB, (k + 1) * GB):
            pltpu.async_copy(t_hbm.at[idx_v.at[r]], vals_v.at[r],
                             sems[k % NSEM])

    def drain_batch(k):
        for _ in range(GB):
            pltpu.make_async_copy(
                t_hbm.at[idx_v.at[0]], vals_v.at[0], sems[k % NSEM]).wait()

    accs = [jnp.zeros((16,), jnp.float32) for _ in range(NGRP)]

    def accum_batch(k, accs):
        out = list(accs)
        for r in range(k * GB, (k + 1) * GB):
            for g in range(NGRP):
                out[g] = out[g] + vals_v[r, pl.ds(g * 16, 16)]
        return out

    for k in range(NSEM):
        fire_batch(k)
    pltpu.sync_copy(
        ids_hbm.at[pl.ds(SPLIT, SEQ - SPLIT), pl.ds(s0, SAMP_PER_W)],
        idx_v.at[pl.ds(SPLIT, SEQ - SPLIT)])
    for k in range(NSEM, NBATCH):
        drain_batch(k - NSEM)
        fire_batch(k)
        accs = accum_batch(k - NSEM, accs)
    for k in range(NBATCH - NSEM, NBATCH):
        drain_batch(k)
        accs = accum_batch(k, accs)

    for g in range(NGRP):
        out_v[pl.ds(g * 16, 16)] = accs[g]

    pltpu.sync_copy(out_v, out_hbm.at[pl.ds(s0, SAMP_PER_W)])


@functools.lru_cache(maxsize=1)
def _make_pool():
    return functools.partial(
        pl.kernel,
        mesh=plsc.VectorSubcoreMesh(core_axis_name="c", subcore_axis_name="s"),
        compiler_params=pltpu.CompilerParams(needs_layout_passes=False),
        out_type=jax.ShapeDtypeStruct((BATCH,), jnp.float32),
        scratch_types=[
            pltpu.VMEM((SEQ, 128), jnp.int32),
            pltpu.VMEM((SEQ, 128), jnp.float32),
            pltpu.VMEM((SAMP_PER_W,), jnp.float32),
            pltpu.SemaphoreType.DMA,
            pltpu.SemaphoreType.DMA,
            pltpu.SemaphoreType.DMA,
        ],
    )(_pool_body)


def kernel(input_ids, embed_table, W, b):
    ids_t = input_ids.astype(jnp.int32).T
    table_t = embed_table.T
    ws = W.astype(jnp.float32) / SEQ
    b2 = (b.astype(jnp.float32) / SEQ).reshape(1, 1)
    t = _compute_t(table_t, ws, b2)
    pooled = _make_pool()(ids_t, t)
    return pooled.reshape(BATCH, 1)


# speedup vs baseline: 1.0474x; 1.0474x over previous
import functools

import jax
import jax.numpy as jnp
from jax import lax
from jax.experimental import pallas as pl
from jax.experimental.pallas import tpu as pltpu
from jax.experimental.pallas import tpu_sc as plsc

VOCAB = 1000000
HIDDEN = 32
BATCH = 4096
SEQ = 200


BN = 131072
NBLK = (VOCAB + BN - 1) // BN


def _matvec_body(x_ref, w_ref, b_ref, o_ref):
    o_ref[...] = jnp.sum(x_ref[...] * w_ref[...], axis=0) + b_ref[0, 0]


def _compute_t(table_t, ws, b2):
    return pl.pallas_call(
        _matvec_body,
        grid=(NBLK,),
        in_specs=[
            pl.BlockSpec((HIDDEN, BN), lambda i: (0, i)),
            pl.BlockSpec((HIDDEN, 1), lambda i: (0, 0)),
            pl.BlockSpec((1, 1), lambda i: (0, 0)),
        ],
        out_specs=pl.BlockSpec((BN,), lambda i: (i,)),
        out_shape=jax.ShapeDtypeStruct((VOCAB,), jnp.float32),
    )(table_t, ws, b2)


NUM_WORKERS = 32
SAMP_PER_W = BATCH // NUM_WORKERS
GB = 25
NBATCH = SEQ // GB
NSEM = 3
NGRP = SAMP_PER_W // 16


def _pool_body(ids_hbm, t_hbm, out_hbm, idx_v, vals_v, out_v,
               sem0, sem1, sem2):
    wid = lax.axis_index("s") * 2 + lax.axis_index("c")
    s0 = wid * SAMP_PER_W
    sems = (sem0, sem1, sem2)

    SPLIT = 80
    pltpu.sync_copy(ids_hbm.at[pl.ds(0, SPLIT), pl.ds(s0, SAMP_PER_W)],
                    idx_v.at[pl.ds(0, SPLIT)])

    def fire_batch(k):
        for r in range(k * GB, (k + 1) * GB):
            pltpu.async_copy(t_hbm.at[idx_v.at[r]], vals_v.at[r],
                             sems[k % NSEM])

    def drain_batch(k):
        for _ in range(GB):
            pltpu.make_async_copy(
                t_hbm.at[idx_v.at[0]], vals_v.at[0], sems[k % NSEM]).wait()

    accs = [jnp.zeros((16,), jnp.float32) for _ in range(NGRP)]

    def accum_batch(k, accs):
        out = list(accs)
        for r in range(k * GB, (k + 1) * GB):
            for g in range(NGRP):
                out[g] = out[g] + vals_v[r, pl.ds(g * 16, 16)]
        return out

    for k in range(NSEM):
        fire_batch(k)
    pltpu.sync_copy(
        ids_hbm.at[pl.ds(SPLIT, SEQ - SPLIT), pl.ds(s0, SAMP_PER_W)],
        idx_v.at[pl.ds(SPLIT, SEQ - SPLIT)])
    for k in range(NSEM, NBATCH):
        drain_batch(k - NSEM)
        fire_batch(k)
        accs = accum_batch(k - NSEM, accs)
    for k in range(NBATCH - NSEM, NBATCH):
        drain_batch(k)
        accs = accum_batch(k, accs)

    for g in range(NGRP):
        out_v[pl.ds(g * 16, 16)] = accs[g]

    pltpu.sync_copy(out_v, out_hbm.at[pl.ds(s0, SAMP_PER_W)])


@functools.lru_cache(maxsize=1)
def _make_pool():
    return functools.partial(
        pl.kernel,
        mesh=plsc.VectorSubcoreMesh(core_axis_name="c", subcore_axis_name="s"),
        compiler_params=pltpu.CompilerParams(needs_layout_passes=False),
        out_type=jax.ShapeDtypeStruct((BATCH,), jnp.float32),
        scratch_types=[
            pltpu.VMEM((SEQ, 128), jnp.int32),
            pltpu.VMEM((SEQ, 128), jnp.float32),
            pltpu.VMEM((SAMP_PER_W,), jnp.float32),
            pltpu.SemaphoreType.DMA,
            pltpu.SemaphoreType.DMA,
            pltpu.SemaphoreType.DMA,
        ],
    )(_pool_body)


def kernel(input_ids, embed_table, W, b):
    ids_t = input_ids.astype(jnp.int32).T
    table_t = embed_table.T
    ws = W.astype(jnp.float32) / SEQ
    b2 = (b.astype(jnp.float32) / SEQ).reshape(1, 1)
    t = _compute_t(table_t, ws, b2)
    pooled = _make_pool()(ids_t, t)
    return pooled.reshape(BATCH, 1)
